# baseline (speedup 1.0000x reference)
import functools

import jax
import jax.numpy as jnp
from jax import lax
from jax.experimental import pallas as pl
from jax.experimental.pallas import tpu as pltpu
from jax.experimental.pallas import tpu_sc as plsc

V = 1_000_000
D = 64
B = 16384
N = 20
NC, NS, L = 2, 16, 16
NW = NC * NS
BPW = B // NW
C = 32
NCH = BPW // C
NPC = C * N
NIR = NPC // 128
TBLK = 8192
TG = (V + TBLK - 1) // TBLK
VPAD = TG * TBLK


def _tr_body(in_ref, out_ref):
    out_ref[:, :D] = in_ref[...].T


def _relayout(table):
    out = pl.pallas_call(
        _tr_body,
        grid=(TG,),
        in_specs=[pl.BlockSpec((D, TBLK), lambda g: (0, g))],
        out_specs=pl.BlockSpec((TBLK, 128), lambda g: (g, 0)),
        out_shape=jax.ShapeDtypeStruct((VPAD, 128), jnp.float32),
    )(table.T)
    return out.reshape(2 * VPAD, D)


def _sc_scores(target, context, negatives, in_embed, out_embed):
    tgt_r = (target * 2).reshape(NW, NCH, C)
    ctx_r = (context * 2).reshape(NW, NCH, C)
    neg_r = (negatives * 2).reshape(NW, NCH * NIR, 128)

    mesh = plsc.VectorSubcoreMesh(core_axis_name="c", subcore_axis_name="s")

    @functools.partial(
        pl.kernel,
        out_type=(
            jax.ShapeDtypeStruct((B,), jnp.float32),
            jax.ShapeDtypeStruct((B * N,), jnp.float32),
        ),
        mesh=mesh,
        compiler_params=pltpu.CompilerParams(
            needs_layout_passes=False, use_tc_tiling_on_sc=False,
            disable_bounds_checks=True),
        scratch_types=[
            pltpu.VMEM((NCH, C), jnp.int32),
            pltpu.VMEM((NCH, C), jnp.int32),
            pltpu.VMEM((NCH * NIR, 128), jnp.int32),
            pltpu.VMEM((2 * C, D), jnp.float32),
            pltpu.VMEM((2 * C, D), jnp.float32),
            pltpu.VMEM((2 * NPC, D), jnp.float32),
            pltpu.VMEM((BPW,), jnp.float32),
            pltpu.VMEM((BPW * N,), jnp.float32),
            pltpu.SemaphoreType.DMA,
            pltpu.SemaphoreType.DMA,
        ],
    )
    def score_kernel(tgt_hbm, ctx_hbm, neg_hbm, inemb, outemb,
                     pos_out, neg_out,
                     tgt_idx, ctx_idx, neg_idx,
                     tgt_rows, ctx_rows, neg_rows,
                     pos_buf, neg_buf, sem0, sem1):
        wid = lax.axis_index("s") * NC + lax.axis_index("c")
        pltpu.sync_copy(tgt_hbm.at[wid], tgt_idx)
        pltpu.sync_copy(ctx_hbm.at[wid], ctx_idx)
        pltpu.sync_copy(neg_hbm.at[wid], neg_idx)
        iota = lax.iota(jnp.int32, L)
        zero = jnp.zeros((L,), jnp.float32)

        def issue(c, p):
            sem = (sem0, sem1)
            for s in (0, 1):
                @pl.when(p == s)
                def _():
                    pltpu.async_copy(inemb.at[tgt_idx.at[c]],
                                     tgt_rows.at[pl.ds(s * C, C)], sem[s])
                    pltpu.async_copy(outemb.at[ctx_idx.at[c]],
                                     ctx_rows.at[pl.ds(s * C, C)], sem[s])
                    for k in range(NIR):
                        pltpu.async_copy(
                            outemb.at[neg_idx.at[c * NIR + k]],
                            neg_rows.at[pl.ds(s * NPC + k * 128, 128)],
                            sem[s])

        def drain(c, p):
            sem = (sem0, sem1)
            for s in (0, 1):
                @pl.when(p == s)
                def _():
                    pltpu.make_async_copy(
                        inemb.at[tgt_idx.at[c]],
                        tgt_rows.at[pl.ds(s * C, C)], sem[s]).wait()
                    pltpu.make_async_copy(
                        outemb.at[ctx_idx.at[c]],
                        ctx_rows.at[pl.ds(s * C, C)], sem[s]).wait()
                    for k in range(NIR):
                        pltpu.make_async_copy(
                            outemb.at[neg_idx.at[c * NIR + k]],
                            neg_rows.at[pl.ds(s * NPC + k * 128, 128)],
                            sem[s]).wait()

        issue(0, 0)

        def chunk_body(c, carry):
            p = lax.rem(c, 2)

            @pl.when(c + 1 < NCH)
            def _():
                issue(c + 1, 1 - p)

            drain(c, p)

            for g in range(C // L):
                local = iota + (g * L + p * C)
                nbase = (iota + g * L) * N + p * NPC
                off = c * C + g * L

                def dbody1(d, acc):
                    pacc, naccs = acc
                    col = jnp.full((L,), d, jnp.int32)
                    t = plsc.load_gather(tgt_rows, [local, col])
                    cx = plsc.load_gather(ctx_rows, [local, col])
                    pacc = pacc + t * cx
                    out = []
                    for j in range(N // 2):
                        nv = plsc.load_gather(neg_rows, [nbase + j, col])
                        out.append(naccs[j] + nv * t)
                    return (pacc, tuple(out))

                pacc, naccs = lax.fori_loop(
                    0, D, dbody1, (zero, (zero,) * (N // 2)), unroll=8)
                pos_buf[pl.ds(off, L)] = pacc
                for j in range(N // 2):
                    neg_buf[pl.ds(j * BPW + off, L)] = naccs[j]

                def dbody2(d, naccs):
                    col = jnp.full((L,), d, jnp.int32)
                    t = plsc.load_gather(tgt_rows, [local, col])
                    out = []
                    for j in range(N // 2, N):
                        nv = plsc.load_gather(neg_rows, [nbase + j, col])
                        out.append(naccs[j - N // 2] + nv * t)
                    return tuple(out)

                naccs = lax.fori_loop(
                    0, D, dbody2, (zero,) * (N // 2), unroll=8)
                for j in range(N // 2, N):
                    neg_buf[pl.ds(j * BPW + off, L)] = naccs[j - N // 2]
            return carry

        lax.fori_loop(0, NCH, chunk_body, 0)
        pltpu.sync_copy(pos_buf, pos_out.at[pl.ds(wid * BPW, BPW)])
        pltpu.sync_copy(neg_buf, neg_out.at[pl.ds(wid * BPW * N, BPW * N)])

    return score_kernel(tgt_r, ctx_r, neg_r,
                        _relayout(in_embed), _relayout(out_embed))


def _loss_body(pos_ref, neg_ref, out_ref):
    s = (jnp.sum(jax.nn.log_sigmoid(pos_ref[...]))
         + jnp.sum(jax.nn.log_sigmoid(-neg_ref[...])))
    out_ref[0, 0] = -s / B


def _loss_tc(pos, neg):
    out = pl.pallas_call(
        _loss_body,
        out_shape=jax.ShapeDtypeStruct((1, 1), jnp.float32),
        out_specs=pl.BlockSpec(memory_space=pltpu.SMEM),
    )(pos.reshape(B // 128, 128), neg.reshape(B * N // 128, 128))
    return out[0, 0]


def kernel(target, context, negatives, in_embed, out_embed):
    pos, neg = _sc_scores(target, context, negatives, in_embed, out_embed)
    return _loss_tc(pos, neg)


# speedup vs baseline: 1.0597x; 1.0597x over previous
import functools

import jax
import jax.numpy as jnp
from jax import lax
from jax.experimental import pallas as pl
from jax.experimental.pallas import tpu as pltpu
from jax.experimental.pallas import tpu_sc as plsc

V = 1_000_000
D = 64
B = 16384
N = 20
NC, NS, L = 2, 16, 16
NW = NC * NS
BPW = B // NW
C = 32
NCH = BPW // C
NPC = C * N
NIR = NPC // 128
TBLK = 8192
TG = (V + TBLK - 1) // TBLK
VPAD = TG * TBLK


def _tr_body(in_ref, out_ref):
    out_ref[:, :D] = in_ref[...].T


def _relayout(table):
    out = pl.pallas_call(
        _tr_body,
        grid=(TG,),
        in_specs=[pl.BlockSpec((D, TBLK), lambda g: (0, g))],
        out_specs=pl.BlockSpec((TBLK, 128), lambda g: (g, 0)),
        out_shape=jax.ShapeDtypeStruct((VPAD, 128), jnp.float32),
    )(table.T)
    return out.reshape(2 * VPAD, D)


def _sc_scores(target, context, negatives, in_embed, out_embed):
    tgt_r = (target * 2).reshape(NW, NCH, C)
    ctx_r = (context * 2).reshape(NW, NCH, C)
    neg_r = (negatives * 2).reshape(NW, NCH * NIR, 128)

    mesh = plsc.VectorSubcoreMesh(core_axis_name="c", subcore_axis_name="s")

    @functools.partial(
        pl.kernel,
        out_type=(
            jax.ShapeDtypeStruct((B,), jnp.float32),
            jax.ShapeDtypeStruct((B * N,), jnp.float32),
        ),
        mesh=mesh,
        compiler_params=pltpu.CompilerParams(
            needs_layout_passes=False, use_tc_tiling_on_sc=False,
            disable_bounds_checks=True),
        scratch_types=[
            pltpu.VMEM((NCH, C), jnp.int32),
            pltpu.VMEM((NCH, C), jnp.int32),
            pltpu.VMEM((NCH * NIR, 128), jnp.int32),
            pltpu.VMEM((2 * C, D), jnp.float32),
            pltpu.VMEM((2 * C, D), jnp.float32),
            pltpu.VMEM((2 * NPC, D), jnp.float32),
            pltpu.VMEM((BPW,), jnp.float32),
            pltpu.VMEM((BPW * N,), jnp.float32),
            pltpu.SemaphoreType.DMA,
            pltpu.SemaphoreType.DMA,
        ],
    )
    def score_kernel(tgt_hbm, ctx_hbm, neg_hbm, inemb, outemb,
                     pos_out, neg_out,
                     tgt_idx, ctx_idx, neg_idx,
                     tgt_rows, ctx_rows, neg_rows,
                     pos_buf, neg_buf, sem0, sem1):
        wid = lax.axis_index("s") * NC + lax.axis_index("c")
        pltpu.sync_copy(tgt_hbm.at[wid], tgt_idx)
        pltpu.sync_copy(ctx_hbm.at[wid], ctx_idx)
        pltpu.sync_copy(neg_hbm.at[wid], neg_idx)
        iota = lax.iota(jnp.int32, L)
        zero = jnp.zeros((L,), jnp.float32)

        def issue(c, p):
            sem = (sem0, sem1)
            for s in (0, 1):
                @pl.when(p == s)
                def _():
                    pltpu.async_copy(inemb.at[tgt_idx.at[c]],
                                     tgt_rows.at[pl.ds(s * C, C)], sem[s])
                    pltpu.async_copy(outemb.at[ctx_idx.at[c]],
                                     ctx_rows.at[pl.ds(s * C, C)], sem[s])
                    for k in range(NIR):
                        pltpu.async_copy(
                            outemb.at[neg_idx.at[c * NIR + k]],
                            neg_rows.at[pl.ds(s * NPC + k * 128, 128)],
                            sem[s])

        def drain(c, p):
            sem = (sem0, sem1)
            for s in (0, 1):
                @pl.when(p == s)
                def _():
                    pltpu.make_async_copy(
                        inemb.at[tgt_idx.at[c]],
                        tgt_rows.at[pl.ds(s * C, C)], sem[s]).wait()
                    pltpu.make_async_copy(
                        outemb.at[ctx_idx.at[c]],
                        ctx_rows.at[pl.ds(s * C, C)], sem[s]).wait()
                    for k in range(NIR):
                        pltpu.make_async_copy(
                            outemb.at[neg_idx.at[c * NIR + k]],
                            neg_rows.at[pl.ds(s * NPC + k * 128, 128)],
                            sem[s]).wait()

        issue(0, 0)

        def chunk_body(c, carry):
            p = lax.rem(c, 2)

            @pl.when(c + 1 < NCH)
            def _():
                issue(c + 1, 1 - p)

            drain(c, p)

            for g in range(C // L):
                local = iota + (g * L + p * C)
                nbase = (iota + g * L) * N + p * NPC
                off = c * C + g * L

                def dbody1(d, acc):
                    pacc, naccs = acc
                    col = jnp.full((L,), d, jnp.int32)
                    t = plsc.load_gather(tgt_rows, [local, col])
                    cx = plsc.load_gather(ctx_rows, [local, col])
                    pacc = pacc + t * cx
                    out = []
                    for j in range(N // 2):
                        nv = plsc.load_gather(neg_rows, [nbase + j, col])
                        out.append(naccs[j] + nv * t)
                    return (pacc, tuple(out))

                pacc, naccs = lax.fori_loop(
                    0, D, dbody1, (zero, (zero,) * (N // 2)), unroll=4)
                pos_buf[pl.ds(off, L)] = pacc
                for j in range(N // 2):
                    neg_buf[pl.ds(j * BPW + off, L)] = naccs[j]

                def dbody2(d, naccs):
                    col = jnp.full((L,), d, jnp.int32)
                    t = plsc.load_gather(tgt_rows, [local, col])
                    out = []
                    for j in range(N // 2, N):
                        nv = plsc.load_gather(neg_rows, [nbase + j, col])
                        out.append(naccs[j - N // 2] + nv * t)
                    return tuple(out)

                naccs = lax.fori_loop(
                    0, D, dbody2, (zero,) * (N // 2), unroll=4)
                for j in range(N // 2, N):
                    neg_buf[pl.ds(j * BPW + off, L)] = naccs[j - N // 2]
            return carry

        lax.fori_loop(0, NCH, chunk_body, 0)
        pltpu.sync_copy(pos_buf, pos_out.at[pl.ds(wid * BPW, BPW)])
        pltpu.sync_copy(neg_buf, neg_out.at[pl.ds(wid * BPW * N, BPW * N)])

    return score_kernel(tgt_r, ctx_r, neg_r,
                        _relayout(in_embed), _relayout(out_embed))


def _loss_body(pos_ref, neg_ref, out_ref):
    s = (jnp.sum(jax.nn.log_sigmoid(pos_ref[...]))
         + jnp.sum(jax.nn.log_sigmoid(-neg_ref[...])))
    out_ref[0, 0] = -s / B


def _loss_tc(pos, neg):
    out = pl.pallas_call(
        _loss_body,
        out_shape=jax.ShapeDtypeStruct((1, 1), jnp.float32),
        out_specs=pl.BlockSpec(memory_space=pltpu.SMEM),
    )(pos.reshape(B // 128, 128), neg.reshape(B * N // 128, 128))
    return out[0, 0]


def kernel(target, context, negatives, in_embed, out_embed):
    pos, neg = _sc_scores(target, context, negatives, in_embed, out_embed)
    return _loss_tc(pos, neg)
